# Initial kernel scaffold; baseline (speedup 1.0000x reference)
import jax
import jax.numpy as jnp
from jax.experimental import pallas as pl


def kernel(x, pos, mask, params):
    raise NotImplementedError("write your pallas kernel here")


# speedup vs baseline: 1.0238x; 1.0238x over previous
import jax
import jax.numpy as jnp
from jax.experimental import pallas as pl
from jax.experimental.pallas import tpu as pltpu

B, N = 2, 2048
D_IN, D_HID, DEPTH, K, M_DIM = 128, 128, 4, 16, 16


def _egnn_layer_jax(p, feats, coors, mask):
    rel_coors_full = coors[:, :, None, :] - coors[:, None, :, :]
    rel_dist_full = jnp.sum(rel_coors_full ** 2, axis=-1)
    pair_valid = mask[:, :, None] & mask[:, None, :]
    ranking = jnp.where(pair_valid, rel_dist_full, 1e5)
    _, nbhd_indices = jax.lax.top_k(-ranking, K)
    rel_coors = jnp.take_along_axis(rel_coors_full, nbhd_indices[..., None], axis=2)
    rel_dist = jnp.take_along_axis(rel_dist_full, nbhd_indices, axis=2)[..., None]
    feats_j = jax.vmap(lambda f, idx: f[idx])(feats, nbhd_indices)
    feats_i = jnp.broadcast_to(feats[:, :, None, :], feats_j.shape)
    edge_input = jnp.concatenate([feats_i, feats_j, rel_dist], axis=-1)
    h = jax.nn.silu(edge_input @ p['edge1'][0] + p['edge1'][1])
    m_ij = jax.nn.silu(h @ p['edge2'][0] + p['edge2'][1])
    mask_j = jax.vmap(lambda m, idx: m[idx])(mask, nbhd_indices)
    pair_mask = mask[:, :, None] & mask_j
    cw = jax.nn.silu(m_ij @ p['coors1'][0] + p['coors1'][1])
    coor_weights = (cw @ p['coors2'][0] + p['coors2'][1])[..., 0]
    norm = jnp.linalg.norm(rel_coors, axis=-1, keepdims=True)
    rel_coors_n = rel_coors / jnp.clip(norm, 1e-8, None) * p['coors_scale']
    coor_weights = jnp.where(pair_mask, coor_weights, 0.0)
    coor_weights = jnp.clip(coor_weights, -2.0, 2.0)
    coors_out = jnp.einsum('bij,bijc->bic', coor_weights, rel_coors_n) + coors
    m_ij = jnp.where(pair_mask[..., None], m_ij, 0.0)
    m_i = jnp.sum(m_ij, axis=2)
    node_in = jnp.concatenate([feats, m_i], axis=-1)
    nh = jax.nn.silu(node_in @ p['node1'][0] + p['node1'][1])
    node_out = nh @ p['node2'][0] + p['node2'][1] + feats
    return node_out, coors_out


def _final_body(h_ref, mf_ref, gw_ref, gb_ref, o_ref):
    b = pl.program_id(0)
    h = h_ref[0]
    mu = jnp.mean(h, axis=-1, keepdims=True)
    var = jnp.mean((h - mu) ** 2, axis=-1, keepdims=True)
    hn = (h - mu) * jax.lax.rsqrt(var + 1e-5) * gw_ref[...] + gb_ref[...]
    mf = mf_ref[0]
    denom = jnp.clip(jnp.sum(mf), 1.0, None)
    o_ref[pl.ds(b, 1), :] = jnp.sum(hn * mf, axis=0, keepdims=True) / denom


def kernel(x, pos, mask, params):
    h = x @ params['in_proj'][0] + params['in_proj'][1]
    c = pos
    for p in params['layers']:
        h, c = _egnn_layer_jax(p, h, c, mask)
    g_w, g_b = params['out_norm']
    maskf = mask.astype(h.dtype)[..., None]
    g = pl.pallas_call(
        _final_body,
        grid=(B,),
        in_specs=[
            pl.BlockSpec((1, N, D_HID), lambda b: (b, 0, 0)),
            pl.BlockSpec((1, N, 1), lambda b: (b, 0, 0)),
            pl.BlockSpec((1, D_HID), lambda b: (0, 0)),
            pl.BlockSpec((1, D_HID), lambda b: (0, 0)),
        ],
        out_specs=pl.BlockSpec((B, D_HID), lambda b: (0, 0)),
        out_shape=jax.ShapeDtypeStruct((B, D_HID), jnp.float32),
    )(h, maskf, g_w.reshape(1, D_HID), g_b.reshape(1, D_HID))
    return g


# speedup vs baseline: 5.9309x; 5.7931x over previous
import functools

import jax
import jax.numpy as jnp
from jax.experimental import pallas as pl
from jax.experimental.pallas import tpu as pltpu

B, N = 2, 2048
D_HID, DEPTH, K, M_DIM = 128, 4, 16, 16
PD = 16
BI = 256
NB = N // BI
E_HID = 514
INF = 3e38


def _inproj_body(x_ref, w_ref, b_ref, o_ref):
    o_ref[...] = x_ref[...] @ w_ref[...] + b_ref[...]


def _in_proj(x2d, w, b):
    return pl.pallas_call(
        _inproj_body,
        grid=(B * N // 512,),
        in_specs=[
            pl.BlockSpec((512, D_HID), lambda i: (i, 0)),
            pl.BlockSpec((D_HID, D_HID), lambda i: (0, 0)),
            pl.BlockSpec((1, D_HID), lambda i: (0, 0)),
        ],
        out_specs=pl.BlockSpec((512, D_HID), lambda i: (i, 0)),
        out_shape=jax.ShapeDtypeStruct((B * N, D_HID), jnp.float32),
    )(x2d, w, b.reshape(1, D_HID))


def _knn_body(pi_ref, pt_ref, o_ref, d_ref):
    b = pl.program_id(0)
    pi = pi_ref[0]
    pt = pt_ref[0]
    xi, yi, zi = pi[:, 0:1], pi[:, 1:2], pi[:, 2:3]
    xj, yj, zj = pt[0:1, :], pt[1:2, :], pt[2:3, :]
    dx = xi - xj
    dy = yi - yj
    dz = zi - zj
    d_ref[...] = dx * dx + dy * dy + dz * dz
    jidx = jax.lax.broadcasted_iota(jnp.int32, (BI, N), 1)
    cols = []
    for _ in range(K):
        d = d_ref[...]
        m = jnp.min(d, axis=1, keepdims=True)
        am = jnp.min(jnp.where(d == m, jidx, N), axis=1, keepdims=True)
        cols.append(am)
        d_ref[...] = jnp.where(jidx == am, INF, d)
    o_ref[0] = jnp.concatenate(cols, axis=1) + b * N


def _knn(pos_pad, pos_t):
    return pl.pallas_call(
        _knn_body,
        grid=(B, NB),
        in_specs=[
            pl.BlockSpec((1, BI, PD), lambda b, i: (b, i, 0)),
            pl.BlockSpec((1, PD, N), lambda b, i: (b, 0, 0)),
        ],
        out_specs=pl.BlockSpec((1, BI, K), lambda b, i: (b, i, 0)),
        out_shape=jax.ShapeDtypeStruct((B, N, K), jnp.int32),
        scratch_shapes=[pltpu.VMEM((BI, N), jnp.float32)],
    )(pos_pad, pos_t)


def _silu(v):
    return v * jax.nn.sigmoid(v)


def _dense_body(fi_ref, gf_ref, gp_ref, pi_ref,
                w1a_ref, w1b_ref, wd_ref, b1_ref, w2_ref, b2_ref,
                wc1_ref, bc1_ref, wc2_ref, bc2_ref, cs_ref,
                wn1a_ref, wn1b_ref, bn1_ref, wn2_ref, bn2_ref,
                fo_ref, po_ref):
    fi = fi_ref[...]
    gf = gf_ref[...]
    gp = gp_ref[...]
    pi = pi_ref[...]

    a = fi @ w1a_ref[...] + b1_ref[...]
    bj = gf @ w1b_ref[...]

    rel3 = pi[:, None, :] - gp.reshape(BI, K, PD)
    d3 = (rel3[:, :, 0] * rel3[:, :, 0]
          + rel3[:, :, 1] * rel3[:, :, 1]
          + rel3[:, :, 2] * rel3[:, :, 2])

    h3 = bj.reshape(BI, K, E_HID) + a[:, None, :] + d3[:, :, None] * wd_ref[...][None, :, :]
    h = _silu(h3).reshape(BI * K, E_HID)
    m = _silu(h @ w2_ref[...] + b2_ref[...])

    cw = _silu(m @ wc1_ref[...] + bc1_ref[...])
    w = cw @ wc2_ref[...] + bc2_ref[...]
    w = jnp.clip(w, -2.0, 2.0)

    d = d3.reshape(BI * K, 1)
    nrm = jnp.sqrt(d)
    reln = rel3.reshape(BI * K, PD) / jnp.maximum(nrm, 1e-8) * cs_ref[0, 0]
    delta = jnp.sum((w * reln).reshape(BI, K, PD), axis=1)
    po_ref[...] = pi + delta

    m_i = jnp.sum(m.reshape(BI, K, M_DIM), axis=1)
    nh = _silu(fi @ wn1a_ref[...] + m_i @ wn1b_ref[...] + bn1_ref[...])
    fo_ref[...] = nh @ wn2_ref[...] + bn2_ref[...] + fi


def _dense(feats, gfeat, gpos, pos_pad2d, wp):
    nblocks = B * N // BI
    grid = (nblocks,)
    w_spec = lambda shape: pl.BlockSpec(shape, lambda i: tuple(0 for _ in shape))
    return pl.pallas_call(
        _dense_body,
        grid=grid,
        in_specs=[
            pl.BlockSpec((BI, D_HID), lambda i: (i, 0)),
            pl.BlockSpec((BI * K, D_HID), lambda i: (i, 0)),
            pl.BlockSpec((BI * K, PD), lambda i: (i, 0)),
            pl.BlockSpec((BI, PD), lambda i: (i, 0)),
            w_spec((D_HID, E_HID)),
            w_spec((D_HID, E_HID)),
            w_spec((1, E_HID)),
            w_spec((1, E_HID)),
            w_spec((E_HID, M_DIM)),
            w_spec((1, M_DIM)),
            w_spec((M_DIM, 4 * M_DIM)),
            w_spec((1, 4 * M_DIM)),
            w_spec((4 * M_DIM, 1)),
            w_spec((1, 1)),
            w_spec((1, 1)),
            w_spec((D_HID, 2 * D_HID)),
            w_spec((M_DIM, 2 * D_HID)),
            w_spec((1, 2 * D_HID)),
            w_spec((2 * D_HID, D_HID)),
            w_spec((1, D_HID)),
        ],
        out_specs=[
            pl.BlockSpec((BI, D_HID), lambda i: (i, 0)),
            pl.BlockSpec((BI, PD), lambda i: (i, 0)),
        ],
        out_shape=[
            jax.ShapeDtypeStruct((B * N, D_HID), jnp.float32),
            jax.ShapeDtypeStruct((B * N, PD), jnp.float32),
        ],
    )(feats, gfeat, gpos, pos_pad2d, *wp)


def _layer_weights(p):
    w1, b1 = p['edge1']
    w2, b2 = p['edge2']
    wc1, bc1 = p['coors1']
    wc2, bc2 = p['coors2']
    wn1, bn1 = p['node1']
    wn2, bn2 = p['node2']
    return (
        w1[:D_HID], w1[D_HID:2 * D_HID], w1[2 * D_HID:2 * D_HID + 1],
        b1.reshape(1, E_HID), w2, b2.reshape(1, M_DIM),
        wc1, bc1.reshape(1, 4 * M_DIM), wc2, bc2.reshape(1, 1),
        p['coors_scale'].reshape(1, 1),
        wn1[:D_HID], wn1[D_HID:], bn1.reshape(1, 2 * D_HID),
        wn2, bn2.reshape(1, D_HID),
    )


def _gather(table, idx_flat):
    return jnp.take(table, idx_flat, axis=0)


def _final_body(h_ref, mf_ref, gw_ref, gb_ref, o_ref):
    b = pl.program_id(0)
    h = h_ref[0]
    mu = jnp.mean(h, axis=-1, keepdims=True)
    var = jnp.mean((h - mu) ** 2, axis=-1, keepdims=True)
    hn = (h - mu) * jax.lax.rsqrt(var + 1e-5) * gw_ref[...] + gb_ref[...]
    mf = mf_ref[0]
    denom = jnp.clip(jnp.sum(mf), 1.0, None)
    o_ref[pl.ds(b, 1), :] = jnp.sum(hn * mf, axis=0, keepdims=True) / denom


def _final(h3d, maskf, g_w, g_b):
    return pl.pallas_call(
        _final_body,
        grid=(B,),
        in_specs=[
            pl.BlockSpec((1, N, D_HID), lambda b: (b, 0, 0)),
            pl.BlockSpec((1, N, 1), lambda b: (b, 0, 0)),
            pl.BlockSpec((1, D_HID), lambda b: (0, 0)),
            pl.BlockSpec((1, D_HID), lambda b: (0, 0)),
        ],
        out_specs=pl.BlockSpec((B, D_HID), lambda b: (0, 0)),
        out_shape=jax.ShapeDtypeStruct((B, D_HID), jnp.float32),
    )(h3d, maskf, g_w.reshape(1, D_HID), g_b.reshape(1, D_HID))


def kernel(x, pos, mask, params):
    x2d = x.reshape(B * N, D_HID)
    feats = _in_proj(x2d, params['in_proj'][0], params['in_proj'][1])
    pos_pad = jnp.pad(pos, ((0, 0), (0, 0), (0, PD - 3)))

    for p in params['layers']:
        pos_t = jnp.swapaxes(pos_pad, 1, 2)
        idx = _knn(pos_pad, pos_t)
        idx_flat = idx.reshape(B * N * K)
        pos2d = pos_pad.reshape(B * N, PD)
        gfeat = _gather(feats, idx_flat)
        gpos = _gather(pos2d, idx_flat)
        feats, pos2d_new = _dense(feats, gfeat, gpos, pos2d, _layer_weights(p))
        pos_pad = pos2d_new.reshape(B, N, PD)

    g_w, g_b = params['out_norm']
    maskf = mask.astype(jnp.float32)[..., None]
    return _final(feats.reshape(B, N, D_HID), maskf, g_w, g_b)


# speedup vs baseline: 9.4457x; 1.5926x over previous
import functools

import jax
import jax.numpy as jnp
from jax import lax
from jax.experimental import pallas as pl
from jax.experimental.pallas import tpu as pltpu
from jax.experimental.pallas import tpu_sc as plsc

B, N = 2, 2048
D_HID, DEPTH, K, M_DIM = 128, 4, 16, 16
PD = 16
BI = 256
NB = N // BI
E_HID = 514
INF = 3e38


def _inproj_body(x_ref, w_ref, b_ref, o_ref):
    o_ref[...] = x_ref[...] @ w_ref[...] + b_ref[...]


def _in_proj(x2d, w, b):
    return pl.pallas_call(
        _inproj_body,
        grid=(B * N // 512,),
        in_specs=[
            pl.BlockSpec((512, D_HID), lambda i: (i, 0)),
            pl.BlockSpec((D_HID, D_HID), lambda i: (0, 0)),
            pl.BlockSpec((1, D_HID), lambda i: (0, 0)),
        ],
        out_specs=pl.BlockSpec((512, D_HID), lambda i: (i, 0)),
        out_shape=jax.ShapeDtypeStruct((B * N, D_HID), jnp.float32),
    )(x2d, w, b.reshape(1, D_HID))


def _knn_body(pi_ref, pt_ref, o_ref, ds_ref, rel_ref, d_ref):
    b = pl.program_id(0)
    pi = pi_ref[0]
    pt = pt_ref[0]
    xi, yi, zi = pi[:, 0:1], pi[:, 1:2], pi[:, 2:3]
    xj, yj, zj = pt[0:1, :], pt[1:2, :], pt[2:3, :]
    dx = xi - xj
    dy = yi - yj
    dz = zi - zj
    d_ref[...] = dx * dx + dy * dy + dz * dz
    jidx = jax.lax.broadcasted_iota(jnp.int32, (BI, N), 1)
    zpad = jnp.zeros((BI, PD - 3), jnp.float32)
    icols, dcols = [], []
    for p in range(K):
        d = d_ref[...]
        m = jnp.min(d, axis=1, keepdims=True)
        am = jnp.min(jnp.where(d == m, jidx, N), axis=1, keepdims=True)
        msk = jidx == am
        icols.append(am)
        dcols.append(m)
        d_ref[...] = jnp.where(msk, INF, d)
        pxj = jnp.min(jnp.where(msk, xj, INF), axis=1, keepdims=True)
        pyj = jnp.min(jnp.where(msk, yj, INF), axis=1, keepdims=True)
        pzj = jnp.min(jnp.where(msk, zj, INF), axis=1, keepdims=True)
        rel_ref[0, :, p, :] = jnp.concatenate(
            [xi - pxj, yi - pyj, zi - pzj, zpad], axis=1)
    o_ref[0] = jnp.concatenate(icols, axis=1) + b * N
    ds_ref[0] = jnp.concatenate(dcols, axis=1)


def _knn(pos_pad, pos_t):
    return pl.pallas_call(
        _knn_body,
        grid=(B, NB),
        in_specs=[
            pl.BlockSpec((1, BI, PD), lambda b, i: (b, i, 0)),
            pl.BlockSpec((1, PD, N), lambda b, i: (b, 0, 0)),
        ],
        out_specs=[
            pl.BlockSpec((1, BI, K), lambda b, i: (b, i, 0)),
            pl.BlockSpec((1, BI, K), lambda b, i: (b, i, 0)),
            pl.BlockSpec((1, BI, K, PD), lambda b, i: (b, i, 0, 0)),
        ],
        out_shape=[
            jax.ShapeDtypeStruct((B, N, K), jnp.int32),
            jax.ShapeDtypeStruct((B, N, K), jnp.float32),
            jax.ShapeDtypeStruct((B, N, K, PD), jnp.float32),
        ],
        scratch_shapes=[pltpu.VMEM((BI, N), jnp.float32)],
    )(pos_pad, pos_t)


def _silu(v):
    return v * jax.nn.sigmoid(v)


def _dense_body(fi_ref, gf_ref, rel_ref, ds_ref, pi_ref,
                w1a_ref, w1b_ref, wd_ref, b1_ref, w2_ref, b2_ref,
                wc1_ref, bc1_ref, wc2_ref, bc2_ref, cs_ref,
                wn1a_ref, wn1b_ref, bn1_ref, wn2_ref, bn2_ref,
                fo_ref, po_ref):
    fi = fi_ref[...]
    gf = gf_ref[...]
    rel3 = rel_ref[...]
    d3 = ds_ref[...]
    pi = pi_ref[...]

    a = fi @ w1a_ref[...] + b1_ref[...]
    bj = gf @ w1b_ref[...]

    h3 = bj.reshape(BI, K, E_HID) + a[:, None, :] + d3[:, :, None] * wd_ref[...][None, :, :]
    h = _silu(h3).reshape(BI * K, E_HID)
    m = _silu(h @ w2_ref[...] + b2_ref[...])

    cw = _silu(m @ wc1_ref[...] + bc1_ref[...])
    w = cw @ wc2_ref[...] + bc2_ref[...]
    w = jnp.clip(w, -2.0, 2.0)

    nrm3 = jnp.maximum(jnp.sqrt(d3), 1e-8)
    reln = (rel3 / nrm3[:, :, None]).reshape(BI * K, PD) * cs_ref[0, 0]
    delta = jnp.sum((w * reln).reshape(BI, K, PD), axis=1)
    po_ref[...] = pi + delta

    m_i = jnp.sum(m.reshape(BI, K, M_DIM), axis=1)
    nh = _silu(fi @ wn1a_ref[...] + m_i @ wn1b_ref[...] + bn1_ref[...])
    fo_ref[...] = nh @ wn2_ref[...] + bn2_ref[...] + fi


def _dense(feats, gfeat, rel, dsel, pos_pad2d, wp):
    nblocks = B * N // BI
    grid = (nblocks,)
    w_spec = lambda shape: pl.BlockSpec(shape, lambda i: tuple(0 for _ in shape))
    return pl.pallas_call(
        _dense_body,
        grid=grid,
        in_specs=[
            pl.BlockSpec((BI, D_HID), lambda i: (i, 0)),
            pl.BlockSpec((BI * K, D_HID), lambda i: (i, 0)),
            pl.BlockSpec((BI, K, PD), lambda i: (i, 0, 0)),
            pl.BlockSpec((BI, K), lambda i: (i, 0)),
            pl.BlockSpec((BI, PD), lambda i: (i, 0)),
            w_spec((D_HID, E_HID)),
            w_spec((D_HID, E_HID)),
            w_spec((1, E_HID)),
            w_spec((1, E_HID)),
            w_spec((E_HID, M_DIM)),
            w_spec((1, M_DIM)),
            w_spec((M_DIM, 4 * M_DIM)),
            w_spec((1, 4 * M_DIM)),
            w_spec((4 * M_DIM, 1)),
            w_spec((1, 1)),
            w_spec((1, 1)),
            w_spec((D_HID, 2 * D_HID)),
            w_spec((M_DIM, 2 * D_HID)),
            w_spec((1, 2 * D_HID)),
            w_spec((2 * D_HID, D_HID)),
            w_spec((1, D_HID)),
        ],
        out_specs=[
            pl.BlockSpec((BI, D_HID), lambda i: (i, 0)),
            pl.BlockSpec((BI, PD), lambda i: (i, 0)),
        ],
        out_shape=[
            jax.ShapeDtypeStruct((B * N, D_HID), jnp.float32),
            jax.ShapeDtypeStruct((B * N, PD), jnp.float32),
        ],
    )(feats, gfeat, rel, dsel, pos_pad2d, *wp)


def _layer_weights(p):
    w1, b1 = p['edge1']
    w2, b2 = p['edge2']
    wc1, bc1 = p['coors1']
    wc2, bc2 = p['coors2']
    wn1, bn1 = p['node1']
    wn2, bn2 = p['node2']
    return (
        w1[:D_HID], w1[D_HID:2 * D_HID], w1[2 * D_HID:2 * D_HID + 1],
        b1.reshape(1, E_HID), w2, b2.reshape(1, M_DIM),
        wc1, bc1.reshape(1, 4 * M_DIM), wc2, bc2.reshape(1, 1),
        p['coors_scale'].reshape(1, 1),
        wn1[:D_HID], wn1[D_HID:], bn1.reshape(1, 2 * D_HID),
        wn2, bn2.reshape(1, D_HID),
    )


ROWS = B * N * K
NW = 32
R_W = ROWS // NW
CH = 128
NCHUNK = R_W // CH

@functools.lru_cache(maxsize=None)
def _sc_gather_fn():
    mesh = plsc.VectorSubcoreMesh(core_axis_name="c", subcore_axis_name="s")

    @functools.partial(
        pl.kernel,
        mesh=mesh,
        out_type=jax.ShapeDtypeStruct((ROWS, D_HID), jnp.float32),
        scratch_types=[
            pltpu.VMEM((NCHUNK, CH), jnp.int32),
            pltpu.VMEM((2, CH, D_HID), jnp.float32),
            pltpu.SemaphoreType.DMA,
            pltpu.SemaphoreType.DMA,
        ],
    )
    def _body(ftab, idxh, gf_hbm, idxv, gfv, sem0, sem1):
        wid = lax.axis_index("s") * 2 + lax.axis_index("c")
        base = wid * R_W
        pltpu.sync_copy(idxh.at[pl.ds(wid * NCHUNK, NCHUNK)], idxv)
        sems = (sem0, sem1)
        prev = None
        for g in range(NCHUNK + 1):
            cur = None
            if g < NCHUNK:
                slot = g % 2
                cur = pltpu.async_copy(ftab.at[idxv.at[g]], gfv.at[slot], sems[slot])
            if prev is not None:
                pslot = (g - 1) % 2
                prev.wait()
                off = base + (g - 1) * CH
                pltpu.sync_copy(gfv.at[pslot], gf_hbm.at[pl.ds(off, CH)])
            prev = cur

    return _body


def _sc_gather(ftab, idx_flat):
    return _sc_gather_fn()(ftab, idx_flat.reshape(ROWS // CH, CH))


def _final_body(h_ref, mf_ref, gw_ref, gb_ref, o_ref):
    b = pl.program_id(0)
    h = h_ref[0]
    mu = jnp.mean(h, axis=-1, keepdims=True)
    var = jnp.mean((h - mu) ** 2, axis=-1, keepdims=True)
    hn = (h - mu) * jax.lax.rsqrt(var + 1e-5) * gw_ref[...] + gb_ref[...]
    mf = mf_ref[0]
    denom = jnp.clip(jnp.sum(mf), 1.0, None)
    o_ref[pl.ds(b, 1), :] = jnp.sum(hn * mf, axis=0, keepdims=True) / denom


def _final(h3d, maskf, g_w, g_b):
    return pl.pallas_call(
        _final_body,
        grid=(B,),
        in_specs=[
            pl.BlockSpec((1, N, D_HID), lambda b: (b, 0, 0)),
            pl.BlockSpec((1, N, 1), lambda b: (b, 0, 0)),
            pl.BlockSpec((1, D_HID), lambda b: (0, 0)),
            pl.BlockSpec((1, D_HID), lambda b: (0, 0)),
        ],
        out_specs=pl.BlockSpec((B, D_HID), lambda b: (0, 0)),
        out_shape=jax.ShapeDtypeStruct((B, D_HID), jnp.float32),
    )(h3d, maskf, g_w.reshape(1, D_HID), g_b.reshape(1, D_HID))


def kernel(x, pos, mask, params):
    x2d = x.reshape(B * N, D_HID)
    feats = _in_proj(x2d, params['in_proj'][0], params['in_proj'][1])
    pos_pad = jnp.pad(pos, ((0, 0), (0, 0), (0, PD - 3)))

    for p in params['layers']:
        pos_t = jnp.swapaxes(pos_pad, 1, 2)
        idx, dsel, rel = _knn(pos_pad, pos_t)
        idx_flat = idx.reshape(B * N * K)
        pos2d = pos_pad.reshape(B * N, PD)
        gfeat = _sc_gather(feats, idx_flat)
        feats, pos2d_new = _dense(feats, gfeat,
                                  rel.reshape(B * N, K, PD),
                                  dsel.reshape(B * N, K),
                                  pos2d, _layer_weights(p))
        pos_pad = pos2d_new.reshape(B, N, PD)

    g_w, g_b = params['out_norm']
    maskf = mask.astype(jnp.float32)[..., None]
    return _final(feats.reshape(B, N, D_HID), maskf, g_w, g_b)
